# baseline (speedup 1.0000x reference)
import jax
import jax.numpy as jnp
from jax import lax
from jax.experimental import pallas as pl
from jax.experimental.pallas import tpu as pltpu
from jax.experimental.pallas import tpu_sc as plsc

HIDDEN = 1024
NUM_CLASSES = 512
NUM_EXPERTS = 8
LOAD_BALANCE_WEIGHT = 0.05
TOKENS = 2048

BM = 256
RBM = 256
EPAD = 128
NUM_BLOCKS = TOKENS // BM
NSLOTS = NUM_BLOCKS + NUM_EXPERTS - 1
NSCHED = 32

NC = 2
NSUB = 16
L = 16
WCHUNK = TOKENS // NSUB
HALF = WCHUNK // NC


def _router_body(x_ref, wr_ref, br_ref, probs_ref, idx_ref, loss_ref,
                 imp_acc, cnt_acc):
    b = pl.program_id(0)
    lanes = lax.broadcasted_iota(jnp.int32, (RBM, EPAD), 1)
    valid = lanes < NUM_EXPERTS
    logits = jnp.dot(x_ref[...], wr_ref[...],
                     preferred_element_type=jnp.float32) + br_ref[...]
    neg = jnp.full_like(logits, -jnp.inf)
    ml = jnp.where(valid, logits, neg)
    m = jnp.max(ml, axis=1, keepdims=True)
    e = jnp.where(valid, jnp.exp(logits - m), 0.0)
    s = jnp.sum(e, axis=1, keepdims=True)
    p = e / s
    probs_ref[...] = p
    idx = jnp.argmax(ml, axis=1, keepdims=True).astype(jnp.int32)
    idx_ref[...] = idx
    onehot = jnp.where(lanes == idx, 1.0, 0.0)

    @pl.when(b == 0)
    def _():
        imp_acc[...] = jnp.zeros_like(imp_acc)
        cnt_acc[...] = jnp.zeros_like(cnt_acc)

    imp_acc[...] += jnp.sum(p, axis=0, keepdims=True)
    cnt_acc[...] += jnp.sum(onehot, axis=0, keepdims=True)

    @pl.when(b == pl.num_programs(0) - 1)
    def _():
        scale = LOAD_BALANCE_WEIGHT * NUM_EXPERTS / (TOKENS * TOKENS)
        loss_ref[...] = scale * jnp.sum(imp_acc[...] * cnt_acc[...],
                                        keepdims=True)


def _sum_splat(x):
    return plsc.cummax(lax.rev(plsc.cumsum(x), (0,)))


def _dispatch_body(idx_hbm, x_hbm,
                   xs_hbm, pos_hbm, grp_hbm, blk_hbm, rs_hbm, re_hbm,
                   idx_all, idx_v, pos_v, x_v, hist_v,
                   t0, t1, t2, t3, sem):
    c = lax.axis_index("c")
    s = lax.axis_index("s")
    lanes = lax.broadcasted_iota(jnp.int32, (L,), 0)
    ones = jnp.ones((L,), jnp.int32)

    pltpu.sync_copy(idx_hbm, idx_all)
    pltpu.sync_copy(idx_hbm.at[pl.ds(s * WCHUNK, WCHUNK)], idx_v)
    row0 = s * WCHUNK + c * HALF
    xcp = pltpu.async_copy(x_hbm.at[pl.ds(row0, HALF)], x_v, sem)

    pre = jnp.zeros((L,), jnp.int32)
    total = jnp.zeros((L,), jnp.int32)
    for w in range(NSUB):
        hist_v[...] = jnp.zeros((L,), jnp.int32)
        for j in range(WCHUNK // L):
            v = idx_all[pl.ds(w * WCHUNK + j * L, L)]
            plsc.addupdate_scatter(hist_v, [v], ones)
        row = hist_v[...]
        before = jnp.full((L,), w, jnp.int32) < s
        pre = pre + jnp.where(before, row, 0)
        total = total + row
    ends = plsc.cumsum(total)
    starts = ends - total
    run = starts + pre

    for j in range(WCHUNK // L):
        v = idx_v[pl.ds(j * L, L)]
        pos = jnp.zeros((L,), jnp.int32)
        for e in range(NUM_EXPERTS):
            m = v == e
            mi = m.astype(jnp.int32)
            incl = plsc.cumsum(mi)
            excl = incl - mi
            cnt = plsc.cummax(lax.rev(incl, (0,)))
            base = _sum_splat(jnp.where(lanes == e, run, 0))
            pos = jnp.where(m, base + excl, pos)
            run = jnp.where(lanes == e, run + cnt, run)
        pos_v[j // (HALF // L), pl.ds((j % (HALF // L)) * L, L)] = pos

    @pl.when(c == 0)
    def _():
        pltpu.sync_copy(pos_v, pos_hbm.at[s])

    xcp.wait()
    pltpu.async_copy(x_v, xs_hbm.at[pos_v.at[c]], sem).wait()

    @pl.when((c == 0) & (s == 0))
    def _():
        fb = starts // BM
        lbm1 = (ends - 1) // BM
        tiles = jnp.where(total > 0, lbm1 - fb + 1, 0)
        incl = plsc.cumsum(tiles)
        off = incl - tiles
        np_ = _sum_splat(tiles)
        t0[...] = starts
        t1[...] = ends
        t2[...] = fb
        t3[...] = off
        for h in range(NSCHED // L):
            slots = lanes + h * L
            g = jnp.zeros((L,), jnp.int32)
            for e in range(NUM_EXPERTS):
                off_e = _sum_splat(jnp.where(lanes == e, off, 0))
                t_e = _sum_splat(jnp.where(lanes == e, tiles, 0))
                g = jnp.where((slots >= off_e) & (t_e > 0), e, g)
            st_g = plsc.load_gather(t0, [g])
            en_g = plsc.load_gather(t1, [g])
            fb_g = plsc.load_gather(t2, [g])
            off_g = plsc.load_gather(t3, [g])
            blk = jnp.clip(fb_g + (slots - off_g), 0, NUM_BLOCKS - 1)
            rs = jnp.maximum(st_g, blk * BM)
            re = jnp.minimum(en_g, (blk + 1) * BM)
            padv = slots >= np_
            rs = jnp.where(padv, 0, rs)
            re = jnp.where(padv, 0, re)
            hist_v[...] = g
            pltpu.sync_copy(hist_v, grp_hbm.at[pl.ds(h * L, L)])
            hist_v[...] = blk
            pltpu.sync_copy(hist_v, blk_hbm.at[pl.ds(h * L, L)])
            hist_v[...] = rs
            pltpu.sync_copy(hist_v, rs_hbm.at[pl.ds(h * L, L)])
            hist_v[...] = re
            pltpu.sync_copy(hist_v, re_hbm.at[pl.ds(h * L, L)])


def _gmm_body(grp_s, blk_s, rs_s, re_s,
              xs_ref, w1_ref, b1_ref, w2_ref, b2_ref, out_ref):
    i = pl.program_id(0)
    rs = rs_s[i]
    re = re_s[i]
    base = blk_s[i] * BM
    rows = base + lax.broadcasted_iota(jnp.int32, (BM, 1), 0)
    mask = (rows >= rs) & (rows < re)
    full = (rs <= base) & (re >= base + BM)

    @pl.when(rs < re)
    def _():
        h = jnp.dot(xs_ref[...], w1_ref[0], preferred_element_type=jnp.float32)
        h = jnp.maximum(h + b1_ref[0], 0.0)
        o = jnp.dot(h, w2_ref[0], preferred_element_type=jnp.float32)
        o = o + b2_ref[0]

        @pl.when(full)
        def _():
            out_ref[...] = o

        @pl.when(jnp.logical_not(full))
        def _():
            out_ref[...] = jnp.where(mask, o, out_ref[...])


def _combine_body(pos_hbm, ys_hbm, out_hbm, pos_half, y_v, sem):
    c = lax.axis_index("c")
    s = lax.axis_index("s")
    pltpu.sync_copy(pos_hbm.at[s, c], pos_half)
    pltpu.async_copy(ys_hbm.at[pos_half], y_v, sem).wait()
    base = s * WCHUNK + c * HALF
    pltpu.sync_copy(y_v, out_hbm.at[pl.ds(base, HALF)])


@jax.jit
def kernel(x, Wr, br, W1, b1, W2, b2):
    wrp = jnp.zeros((HIDDEN, EPAD), jnp.float32).at[:, :NUM_EXPERTS].set(Wr)
    brp = jnp.zeros((1, EPAD), jnp.float32).at[0, :NUM_EXPERTS].set(br)

    probs_pad, idx_col, loss11 = pl.pallas_call(
        _router_body,
        grid=(TOKENS // RBM,),
        in_specs=[
            pl.BlockSpec((RBM, HIDDEN), lambda b: (b, 0)),
            pl.BlockSpec((HIDDEN, EPAD), lambda b: (0, 0)),
            pl.BlockSpec((1, EPAD), lambda b: (0, 0)),
        ],
        out_specs=[
            pl.BlockSpec((RBM, EPAD), lambda b: (b, 0)),
            pl.BlockSpec((RBM, 1), lambda b: (b, 0)),
            pl.BlockSpec((1, 1), lambda b: (0, 0)),
        ],
        out_shape=[
            jax.ShapeDtypeStruct((TOKENS, EPAD), jnp.float32),
            jax.ShapeDtypeStruct((TOKENS, 1), jnp.int32),
            jax.ShapeDtypeStruct((1, 1), jnp.float32),
        ],
        scratch_shapes=[
            pltpu.VMEM((1, EPAD), jnp.float32),
            pltpu.VMEM((1, EPAD), jnp.float32),
        ],
    )(x, wrp, brp)

    mesh = plsc.VectorSubcoreMesh(core_axis_name="c", subcore_axis_name="s")
    xs, pos3, grp, blk, rsv, rev = pl.kernel(
        _dispatch_body,
        out_type=[
            jax.ShapeDtypeStruct((TOKENS, HIDDEN), jnp.float32),
            jax.ShapeDtypeStruct((NSUB, NC, HALF), jnp.int32),
            jax.ShapeDtypeStruct((NSCHED,), jnp.int32),
            jax.ShapeDtypeStruct((NSCHED,), jnp.int32),
            jax.ShapeDtypeStruct((NSCHED,), jnp.int32),
            jax.ShapeDtypeStruct((NSCHED,), jnp.int32),
        ],
        mesh=mesh,
        compiler_params=pltpu.CompilerParams(needs_layout_passes=False),
        scratch_types=[
            pltpu.VMEM((TOKENS,), jnp.int32),
            pltpu.VMEM((WCHUNK,), jnp.int32),
            pltpu.VMEM((NC, HALF), jnp.int32),
            pltpu.VMEM((HALF, HIDDEN), jnp.float32),
            pltpu.VMEM((L,), jnp.int32),
            pltpu.VMEM((L,), jnp.int32),
            pltpu.VMEM((L,), jnp.int32),
            pltpu.VMEM((L,), jnp.int32),
            pltpu.VMEM((L,), jnp.int32),
            pltpu.SemaphoreType.DMA,
        ],
    )(idx_col.reshape(TOKENS), x)

    ys = pl.pallas_call(
        _gmm_body,
        grid_spec=pltpu.PrefetchScalarGridSpec(
            num_scalar_prefetch=4,
            grid=(NSLOTS,),
            in_specs=[
                pl.BlockSpec((BM, HIDDEN), lambda i, g, b, r, e: (b[i], 0)),
                pl.BlockSpec((1, HIDDEN, HIDDEN),
                             lambda i, g, b, r, e: (g[i], 0, 0)),
                pl.BlockSpec((1, 1, HIDDEN), lambda i, g, b, r, e: (g[i], 0, 0)),
                pl.BlockSpec((1, HIDDEN, NUM_CLASSES),
                             lambda i, g, b, r, e: (g[i], 0, 0)),
                pl.BlockSpec((1, 1, NUM_CLASSES),
                             lambda i, g, b, r, e: (g[i], 0, 0)),
            ],
            out_specs=pl.BlockSpec((BM, NUM_CLASSES),
                                   lambda i, g, b, r, e: (b[i], 0)),
        ),
        out_shape=jax.ShapeDtypeStruct((TOKENS, NUM_CLASSES), jnp.float32),
    )(grp, blk, rsv, rev, xs, W1, b1.reshape(NUM_EXPERTS, 1, HIDDEN), W2,
      b2.reshape(NUM_EXPERTS, 1, NUM_CLASSES))

    logits = pl.kernel(
        _combine_body,
        out_type=jax.ShapeDtypeStruct((TOKENS, NUM_CLASSES), jnp.float32),
        mesh=plsc.VectorSubcoreMesh(core_axis_name="c", subcore_axis_name="s"),
        compiler_params=pltpu.CompilerParams(needs_layout_passes=False),
        scratch_types=[
            pltpu.VMEM((HALF,), jnp.int32),
            pltpu.VMEM((HALF, NUM_CLASSES), jnp.float32),
            pltpu.SemaphoreType.DMA,
        ],
    )(pos3, ys)

    return logits, loss11[0, 0], probs_pad[:, :NUM_EXPERTS]


# speedup vs baseline: 1.0386x; 1.0386x over previous
import jax
import jax.numpy as jnp
from jax import lax
from jax.experimental import pallas as pl
from jax.experimental.pallas import tpu as pltpu
from jax.experimental.pallas import tpu_sc as plsc

HIDDEN = 1024
NUM_CLASSES = 512
NUM_EXPERTS = 8
LOAD_BALANCE_WEIGHT = 0.05
TOKENS = 2048

BM = 256
RBM = 256
EPAD = 128
NUM_BLOCKS = TOKENS // BM
NSLOTS = NUM_BLOCKS + NUM_EXPERTS - 1
NSCHED = 32

NC = 2
NSUB = 16
L = 16
WCHUNK = TOKENS // NSUB
HALF = WCHUNK // NC


def _router_body(x_ref, wr_ref, br_ref, probs_ref, idx_ref, loss_ref,
                 imp_acc, cnt_acc):
    b = pl.program_id(0)
    lanes = lax.broadcasted_iota(jnp.int32, (RBM, NUM_EXPERTS), 1)
    logits = jnp.dot(x_ref[...], wr_ref[...],
                     preferred_element_type=jnp.float32) + br_ref[...]
    m = jnp.max(logits, axis=1, keepdims=True)
    e = jnp.exp(logits - m)
    s = jnp.sum(e, axis=1, keepdims=True)
    p = e / s
    probs_ref[...] = p
    idx = jnp.argmax(logits, axis=1, keepdims=True).astype(jnp.int32)
    idx_ref[...] = idx
    onehot = jnp.where(lanes == idx, 1.0, 0.0)

    @pl.when(b == 0)
    def _():
        imp_acc[...] = jnp.zeros_like(imp_acc)
        cnt_acc[...] = jnp.zeros_like(cnt_acc)

    imp_acc[...] += jnp.sum(p, axis=0, keepdims=True)
    cnt_acc[...] += jnp.sum(onehot, axis=0, keepdims=True)

    @pl.when(b == pl.num_programs(0) - 1)
    def _():
        scale = LOAD_BALANCE_WEIGHT * NUM_EXPERTS / (TOKENS * TOKENS)
        loss_ref[...] = scale * jnp.sum(imp_acc[...] * cnt_acc[...],
                                        keepdims=True)


def _sum_splat(x):
    return plsc.cummax(lax.rev(plsc.cumsum(x), (0,)))


def _dispatch_body(idx_hbm, x_hbm,
                   xs_hbm, pos_hbm, grp_hbm, blk_hbm, rs_hbm, re_hbm,
                   idx_all, idx_v, pos_v, x_v, hist_v,
                   t0, t1, t2, t3, sem):
    c = lax.axis_index("c")
    s = lax.axis_index("s")
    lanes = lax.broadcasted_iota(jnp.int32, (L,), 0)
    ones = jnp.ones((L,), jnp.int32)

    pltpu.sync_copy(idx_hbm, idx_all)
    pltpu.sync_copy(idx_hbm.at[pl.ds(s * WCHUNK, WCHUNK)], idx_v)
    row0 = s * WCHUNK + c * HALF
    xcp = pltpu.async_copy(x_hbm.at[pl.ds(row0, HALF)], x_v, sem)

    pre = jnp.zeros((L,), jnp.int32)
    total = jnp.zeros((L,), jnp.int32)
    for w in range(NSUB):
        hist_v[...] = jnp.zeros((L,), jnp.int32)
        for j in range(WCHUNK // L):
            v = idx_all[pl.ds(w * WCHUNK + j * L, L)]
            plsc.addupdate_scatter(hist_v, [v], ones)
        row = hist_v[...]
        before = jnp.full((L,), w, jnp.int32) < s
        pre = pre + jnp.where(before, row, 0)
        total = total + row
    ends = plsc.cumsum(total)
    starts = ends - total
    run = starts + pre

    for j in range(WCHUNK // L):
        v = idx_v[pl.ds(j * L, L)]
        pos = jnp.zeros((L,), jnp.int32)
        for e in range(NUM_EXPERTS):
            m = v == e
            mi = m.astype(jnp.int32)
            incl = plsc.cumsum(mi)
            excl = incl - mi
            cnt = plsc.cummax(lax.rev(incl, (0,)))
            base = _sum_splat(jnp.where(lanes == e, run, 0))
            pos = jnp.where(m, base + excl, pos)
            run = jnp.where(lanes == e, run + cnt, run)
        pos_v[j // (HALF // L), pl.ds((j % (HALF // L)) * L, L)] = pos

    @pl.when(c == 0)
    def _():
        pltpu.sync_copy(pos_v, pos_hbm.at[s])

    xcp.wait()
    pltpu.async_copy(x_v, xs_hbm.at[pos_v.at[c]], sem).wait()

    @pl.when((c == 0) & (s == 0))
    def _():
        fb = starts // BM
        lbm1 = (ends - 1) // BM
        tiles = jnp.where(total > 0, lbm1 - fb + 1, 0)
        incl = plsc.cumsum(tiles)
        off = incl - tiles
        np_ = _sum_splat(tiles)
        t0[...] = starts
        t1[...] = ends
        t2[...] = fb
        t3[...] = off
        for h in range(NSCHED // L):
            slots = lanes + h * L
            g = jnp.zeros((L,), jnp.int32)
            for e in range(NUM_EXPERTS):
                off_e = _sum_splat(jnp.where(lanes == e, off, 0))
                t_e = _sum_splat(jnp.where(lanes == e, tiles, 0))
                g = jnp.where((slots >= off_e) & (t_e > 0), e, g)
            st_g = plsc.load_gather(t0, [g])
            en_g = plsc.load_gather(t1, [g])
            fb_g = plsc.load_gather(t2, [g])
            off_g = plsc.load_gather(t3, [g])
            blk = jnp.clip(fb_g + (slots - off_g), 0, NUM_BLOCKS - 1)
            rs = jnp.maximum(st_g, blk * BM)
            re = jnp.minimum(en_g, (blk + 1) * BM)
            padv = slots >= np_
            rs = jnp.where(padv, 0, rs)
            re = jnp.where(padv, 0, re)
            hist_v[...] = g
            pltpu.sync_copy(hist_v, grp_hbm.at[pl.ds(h * L, L)])
            hist_v[...] = blk
            pltpu.sync_copy(hist_v, blk_hbm.at[pl.ds(h * L, L)])
            hist_v[...] = rs
            pltpu.sync_copy(hist_v, rs_hbm.at[pl.ds(h * L, L)])
            hist_v[...] = re
            pltpu.sync_copy(hist_v, re_hbm.at[pl.ds(h * L, L)])


def _gmm_body(grp_s, blk_s, rs_s, re_s,
              xs_ref, w1_ref, b1_ref, w2_ref, b2_ref, out_ref):
    i = pl.program_id(0)
    rs = rs_s[i]
    re = re_s[i]
    base = blk_s[i] * BM
    rows = base + lax.broadcasted_iota(jnp.int32, (BM, 1), 0)
    mask = (rows >= rs) & (rows < re)
    full = (rs <= base) & (re >= base + BM)

    @pl.when(rs < re)
    def _():
        h = jnp.dot(xs_ref[...], w1_ref[0], preferred_element_type=jnp.float32)
        h = jnp.maximum(h + b1_ref[0], 0.0)
        o = jnp.dot(h, w2_ref[0], preferred_element_type=jnp.float32)
        o = o + b2_ref[0]

        @pl.when(full)
        def _():
            out_ref[...] = o

        @pl.when(jnp.logical_not(full))
        def _():
            out_ref[...] = jnp.where(mask, o, out_ref[...])


def _combine_body(pos_hbm, ys_hbm, out_hbm, pos_half, y_v, sem):
    c = lax.axis_index("c")
    s = lax.axis_index("s")
    pltpu.sync_copy(pos_hbm.at[s, c], pos_half)
    pltpu.async_copy(ys_hbm.at[pos_half], y_v, sem).wait()
    base = s * WCHUNK + c * HALF
    pltpu.sync_copy(y_v, out_hbm.at[pl.ds(base, HALF)])


@jax.jit
def kernel(x, Wr, br, W1, b1, W2, b2):
    probs, idx_col, loss11 = pl.pallas_call(
        _router_body,
        grid=(TOKENS // RBM,),
        in_specs=[
            pl.BlockSpec((RBM, HIDDEN), lambda b: (b, 0)),
            pl.BlockSpec((HIDDEN, NUM_EXPERTS), lambda b: (0, 0)),
            pl.BlockSpec((1, NUM_EXPERTS), lambda b: (0, 0)),
        ],
        out_specs=[
            pl.BlockSpec((RBM, NUM_EXPERTS), lambda b: (b, 0)),
            pl.BlockSpec((RBM, 1), lambda b: (b, 0)),
            pl.BlockSpec((1, 1), lambda b: (0, 0)),
        ],
        out_shape=[
            jax.ShapeDtypeStruct((TOKENS, NUM_EXPERTS), jnp.float32),
            jax.ShapeDtypeStruct((TOKENS, 1), jnp.int32),
            jax.ShapeDtypeStruct((1, 1), jnp.float32),
        ],
        scratch_shapes=[
            pltpu.VMEM((1, NUM_EXPERTS), jnp.float32),
            pltpu.VMEM((1, NUM_EXPERTS), jnp.float32),
        ],
    )(x, Wr, br.reshape(1, NUM_EXPERTS))

    mesh = plsc.VectorSubcoreMesh(core_axis_name="c", subcore_axis_name="s")
    xs, pos3, grp, blk, rsv, rev = pl.kernel(
        _dispatch_body,
        out_type=[
            jax.ShapeDtypeStruct((TOKENS, HIDDEN), jnp.float32),
            jax.ShapeDtypeStruct((NSUB, NC, HALF), jnp.int32),
            jax.ShapeDtypeStruct((NSCHED,), jnp.int32),
            jax.ShapeDtypeStruct((NSCHED,), jnp.int32),
            jax.ShapeDtypeStruct((NSCHED,), jnp.int32),
            jax.ShapeDtypeStruct((NSCHED,), jnp.int32),
        ],
        mesh=mesh,
        compiler_params=pltpu.CompilerParams(needs_layout_passes=False),
        scratch_types=[
            pltpu.VMEM((TOKENS,), jnp.int32),
            pltpu.VMEM((WCHUNK,), jnp.int32),
            pltpu.VMEM((NC, HALF), jnp.int32),
            pltpu.VMEM((HALF, HIDDEN), jnp.float32),
            pltpu.VMEM((L,), jnp.int32),
            pltpu.VMEM((L,), jnp.int32),
            pltpu.VMEM((L,), jnp.int32),
            pltpu.VMEM((L,), jnp.int32),
            pltpu.VMEM((L,), jnp.int32),
            pltpu.SemaphoreType.DMA,
        ],
    )(idx_col.reshape(TOKENS), x)

    ys = pl.pallas_call(
        _gmm_body,
        grid_spec=pltpu.PrefetchScalarGridSpec(
            num_scalar_prefetch=4,
            grid=(NSLOTS,),
            in_specs=[
                pl.BlockSpec((BM, HIDDEN), lambda i, g, b, r, e: (b[i], 0)),
                pl.BlockSpec((1, HIDDEN, HIDDEN),
                             lambda i, g, b, r, e: (g[i], 0, 0)),
                pl.BlockSpec((1, 1, HIDDEN), lambda i, g, b, r, e: (g[i], 0, 0)),
                pl.BlockSpec((1, HIDDEN, NUM_CLASSES),
                             lambda i, g, b, r, e: (g[i], 0, 0)),
                pl.BlockSpec((1, 1, NUM_CLASSES),
                             lambda i, g, b, r, e: (g[i], 0, 0)),
            ],
            out_specs=pl.BlockSpec((BM, NUM_CLASSES),
                                   lambda i, g, b, r, e: (b[i], 0)),
        ),
        out_shape=jax.ShapeDtypeStruct((TOKENS, NUM_CLASSES), jnp.float32),
    )(grp, blk, rsv, rev, xs, W1, b1.reshape(NUM_EXPERTS, 1, HIDDEN), W2,
      b2.reshape(NUM_EXPERTS, 1, NUM_CLASSES))

    logits = pl.kernel(
        _combine_body,
        out_type=jax.ShapeDtypeStruct((TOKENS, NUM_CLASSES), jnp.float32),
        mesh=plsc.VectorSubcoreMesh(core_axis_name="c", subcore_axis_name="s"),
        compiler_params=pltpu.CompilerParams(needs_layout_passes=False),
        scratch_types=[
            pltpu.VMEM((HALF,), jnp.int32),
            pltpu.VMEM((HALF, NUM_CLASSES), jnp.float32),
            pltpu.SemaphoreType.DMA,
        ],
    )(pos3, ys)

    return logits, loss11[0, 0], probs
